# baseline (speedup 1.0000x reference)
import functools

import jax
import jax.numpy as jnp
from jax import lax
from jax.experimental import pallas as pl
from jax.experimental.pallas import tpu as pltpu
from jax.experimental.pallas import tpu_sc as plsc

N = 10000
NP = 10240
E = 320000
C = 128

NC = 2
NS = 16
H = C // NC
B = 112
K = 180
EP = NS * K * B
P = K // 4
ZB = 80
RPS = NP // NS

_mesh = plsc.VectorSubcoreMesh(
    core_axis_name="c", subcore_axis_name="s", num_cores=NC, num_subcores=NS
)


@functools.partial(
    pl.kernel,
    out_type=(
        jax.ShapeDtypeStruct((NC, NP, H), jnp.float32),
        jax.ShapeDtypeStruct((NC, NP, H), jnp.bfloat16),
        jax.ShapeDtypeStruct((NC, NP, H), jnp.bfloat16),
    ),
    mesh=_mesh,
    compiler_params=pltpu.CompilerParams(use_tc_tiling_on_sc=False,
                                         needs_layout_passes=False),
    scratch_types=[
        pltpu.VMEM((K, B), jnp.int32),
        pltpu.VMEM((K, B), jnp.int32),
        pltpu.VMEM((K, B), jnp.float32),
        pltpu.VMEM((B, H), jnp.bfloat16),
        pltpu.VMEM((B, H), jnp.bfloat16),
        pltpu.VMEM((B, H), jnp.bfloat16),
        pltpu.VMEM((B, H), jnp.bfloat16),
        pltpu.VMEM((B, H), jnp.float32),
        pltpu.VMEM((B, H), jnp.float32),
        pltpu.VMEM_SHARED((NP, H), jnp.float32),
        pltpu.SemaphoreType.DMA,
        pltpu.SemaphoreType.DMA,
        pltpu.SemaphoreType.DMA,
        pltpu.SemaphoreType.DMA,
        pltpu.SemaphoreType.DMA,
        pltpu.SemaphoreType.DMA,
        pltpu.SemaphoreType.DMA,
    ],
)
def _sc_prop(table, cols3, rows3, vals3, out, t1b, t0b, cols_v, rows_v, vals_v,
             gbuf0, gbuf1, gbuf2, gbuf3, sbuf0, sbuf1, acc,
             gsem0, gsem1, gsem2, gsem3, ssem0, ssem1, csem):
    c = lax.axis_index("c")
    s = lax.axis_index("s")

    gbufs = (gbuf0, gbuf1, gbuf2, gbuf3)
    sbufs = (sbuf0, sbuf1)
    gsems = (gsem0, gsem1, gsem2, gsem3)
    ssems = (ssem0, ssem1)

    pltpu.sync_copy(cols3.at[s], cols_v)
    pltpu.sync_copy(rows3.at[s], rows_v)
    pltpu.sync_copy(vals3.at[s], vals_v)

    base_row = s * RPS

    def convert(load_chunk, dst):
        for off in range(0, RPS, ZB):
            load_chunk(off, sbuf0)

            def row_body(r, _):
                for m in range(H // 32):
                    a = sbuf0[r, pl.ds(32 * m, 16)]
                    b = sbuf0[r, pl.ds(32 * m + 16, 16)]
                    pk = plsc.pack(a, b, format=plsc.PackFormat.INTERLEAVED)
                    gbuf0[r, pl.ds(32 * m, 32)] = pk
                return 0

            lax.fori_loop(0, ZB, row_body, 0)
            pltpu.sync_copy(gbuf0.at[pl.ds(0, ZB)],
                            dst.at[pl.ds(base_row + off, ZB)])

    def zero_sbuf0():
        z16 = jnp.zeros((16,), jnp.float32)

        def zero_body(i, _):
            for m in range(H // 16):
                sbuf0[i, pl.ds(16 * m, 16)] = z16
            return 0

        lax.fori_loop(0, B, zero_body, 0)

    def zero_acc():
        for off in range(0, RPS, ZB):
            pltpu.sync_copy(sbuf0.at[pl.ds(0, ZB)],
                            acc.at[pl.ds(base_row + off, ZB)])

    def gather_start(src, k, j):
        pltpu.async_copy(src.at[cols_v.at[k]], gbufs[j], gsems[j])

    def gather_wait(j):
        pltpu.make_async_copy(t1b.at[0].at[pl.ds(0, B)], gbufs[j],
                              gsems[j]).wait()

    def scatter_start(k, sb):
        pltpu.async_copy(sbufs[sb], acc.at[rows_v.at[k]], ssems[sb], add=True)

    def scatter_wait(sb):
        pltpu.make_async_copy(sbufs[sb], acc.at[pl.ds(0, B)], ssems[sb]).wait()

    def scale(k, j, sb):
        gbuf = gbufs[j]
        sbuf = sbufs[sb]

        @plsc.parallel_loop(0, B // 16, unroll=2)
        def _group(g):
            vvec = vals_v[k, pl.ds(g * 16, 16)]
            for e in range(16):
                row = g * 16 + e
                v16 = lax.gather(
                    vvec, jnp.full((16, 1), e, jnp.int32),
                    lax.GatherDimensionNumbers(
                        offset_dims=(), collapsed_slice_dims=(0,),
                        start_index_map=(0,)),
                    (1,), mode=lax.GatherScatterMode.PROMISE_IN_BOUNDS)
                for m in range(H // 32):
                    pk = gbuf[row, pl.ds(32 * m, 32)]
                    a, b = plsc.unpack(pk, format=plsc.PackFormat.INTERLEAVED)
                    sbuf[row, pl.ds(32 * m, 16)] = a * v16
                    sbuf[row, pl.ds(32 * m + 16, 16)] = b * v16

    def run_round(src):
        gather_start(src, 0, 0)
        gather_start(src, 1, 1)

        def pipe_body(i, _):
            k0 = 4 * i
            for j in range(4):
                k = k0 + j
                sb = j % 2
                nxt = (j + 2) % 4
                if j < 2:
                    gather_start(src, k + 2, nxt)
                else:
                    pl.when(i < P - 1)(
                        lambda src=src, k=k, nxt=nxt: gather_start(src, k + 2, nxt))
                gather_wait(j)
                if j < 2:
                    pl.when(i > 0)(lambda sb=sb: scatter_wait(sb))
                else:
                    scatter_wait(sb)
                scale(k, j, sb)
                scatter_start(k, sb)
            return 0

        lax.fori_loop(0, P, pipe_body, 0)
        scatter_wait(0)
        scatter_wait(1)

    def drain(dst):
        for off in range(0, RPS, ZB):
            pltpu.sync_copy(acc.at[pl.ds(base_row + off, ZB)],
                            dst.at[pl.ds(base_row + off, ZB)])

    def load_t0(off, fb):
        pltpu.sync_copy(table.at[c].at[pl.ds(base_row + off, ZB)],
                        fb.at[pl.ds(0, ZB)])

    convert(load_t0, t0b.at[c])
    zero_sbuf0()
    zero_acc()
    plsc.subcore_barrier()
    run_round(t0b.at[c])
    plsc.subcore_barrier()

    def load_acc(off, fb):
        pltpu.sync_copy(acc.at[pl.ds(base_row + off, ZB)],
                        fb.at[pl.ds(0, ZB)])

    convert(load_acc, t1b.at[c])
    zero_sbuf0()
    zero_acc()
    plsc.subcore_barrier()
    run_round(t1b.at[c])
    plsc.subcore_barrier()
    drain(out.at[c])


_BLK = 640


def _mm_body(x_ref, w_ref, o_ref):
    d = jnp.dot(x_ref[...], w_ref[...], preferred_element_type=jnp.float32)
    o_ref[0] = d[:, :H]
    o_ref[1] = d[:, H:]


_mm = pl.pallas_call(
    _mm_body,
    grid=(NP // _BLK,),
    in_specs=[
        pl.BlockSpec((_BLK, C), lambda i: (i, 0)),
        pl.BlockSpec((C, C), lambda i: (0, 0)),
    ],
    out_specs=pl.BlockSpec((NC, _BLK, H), lambda i: (0, i, 0)),
    out_shape=jax.ShapeDtypeStruct((NC, NP, H), jnp.float32),
)


def _fin_body(q_ref, bias_ref, o_ref):
    o_ref[...] = (jnp.concatenate([q_ref[0], q_ref[1]], axis=-1)
                  + bias_ref[...])


_fin = pl.pallas_call(
    _fin_body,
    grid=(NP // _BLK,),
    in_specs=[
        pl.BlockSpec((NC, _BLK, H), lambda i: (0, i, 0)),
        pl.BlockSpec((1, C), lambda i: (0, 0)),
    ],
    out_specs=pl.BlockSpec((_BLK, C), lambda i: (i, 0)),
    out_shape=jax.ShapeDtypeStruct((NP, C), jnp.float32),
)


_PB = 128
_ER = E // _PB
_EPR = EP // _PB


def _prep_body(adj_ref, vals_ref, ro_ref, co_ref, vo_ref):
    zi = jnp.zeros((_EPR - _ER, _PB), jnp.int32)
    zf = jnp.zeros((_EPR - _ER, _PB), jnp.float32)
    ro_ref[...] = jnp.concatenate([adj_ref[0], zi], axis=0)
    co_ref[...] = jnp.concatenate([adj_ref[1], zi], axis=0)
    vo_ref[...] = jnp.concatenate([vals_ref[...], zf], axis=0)


_prep = pl.pallas_call(
    _prep_body,
    grid=(1,),
    in_specs=[
        pl.BlockSpec((2, _ER, _PB), lambda i: (0, 0, 0)),
        pl.BlockSpec((_ER, _PB), lambda i: (0, 0)),
    ],
    out_specs=[
        pl.BlockSpec((_EPR, _PB), lambda i: (0, 0)),
        pl.BlockSpec((_EPR, _PB), lambda i: (0, 0)),
        pl.BlockSpec((_EPR, _PB), lambda i: (0, 0)),
    ],
    out_shape=[
        jax.ShapeDtypeStruct((_EPR, _PB), jnp.int32),
        jax.ShapeDtypeStruct((_EPR, _PB), jnp.int32),
        jax.ShapeDtypeStruct((_EPR, _PB), jnp.float32),
    ],
)


def kernel(adj_index, adj_values, features, weight_matrix, bias):
    adj2 = adj_index.astype(jnp.int32).reshape(2, _ER, _PB)
    vals2 = adj_values.astype(jnp.float32).reshape(_ER, _PB)
    rows2, cols2, vals2p = _prep(adj2, vals2)
    rows3 = rows2.reshape(NS, K, B)
    cols3 = cols2.reshape(NS, K, B)
    vals3 = vals2p.reshape(NS, K, B)

    feats_p = jnp.concatenate(
        [features, jnp.zeros((NP - N, C), jnp.float32)], axis=0)
    t = _mm(feats_p, weight_matrix)
    q, _, _ = _sc_prop(t, cols3, rows3, vals3)
    return _fin(q, bias)[:N]


# speedup vs baseline: 1.0921x; 1.0921x over previous
import functools

import jax
import jax.numpy as jnp
from jax import lax
from jax.experimental import pallas as pl
from jax.experimental.pallas import tpu as pltpu
from jax.experimental.pallas import tpu_sc as plsc

N = 10000
NP = 10240
E = 320000
C = 128

NC = 2
NS = 16
H = C // NC
B = 112
K = 180
EP = NS * K * B
P = K // 4
ZB = 80
RPS = NP // NS

_mesh = plsc.VectorSubcoreMesh(
    core_axis_name="c", subcore_axis_name="s", num_cores=NC, num_subcores=NS
)


@functools.partial(
    pl.kernel,
    out_type=(
        jax.ShapeDtypeStruct((NC, NP, H), jnp.float32),
        jax.ShapeDtypeStruct((NC, NP, H), jnp.float32),
    ),
    mesh=_mesh,
    compiler_params=pltpu.CompilerParams(use_tc_tiling_on_sc=False),
    scratch_types=[
        pltpu.VMEM((K, B), jnp.int32),
        pltpu.VMEM((K, B), jnp.int32),
        pltpu.VMEM((K, B), jnp.float32),
        pltpu.VMEM((B, H), jnp.float32),
        pltpu.VMEM((B, H), jnp.float32),
        pltpu.VMEM((B, H), jnp.float32),
        pltpu.VMEM((B, H), jnp.float32),
        pltpu.VMEM_SHARED((NP, H), jnp.float32),
        pltpu.SemaphoreType.DMA,
        pltpu.SemaphoreType.DMA,
        pltpu.SemaphoreType.DMA,
        pltpu.SemaphoreType.DMA,
        pltpu.SemaphoreType.DMA,
        pltpu.SemaphoreType.DMA,
        pltpu.SemaphoreType.DMA,
        pltpu.SemaphoreType.DMA,
    ],
)
def _sc_prop(table, cols3, rows3, vals3, out, t1, cols_v, rows_v, vals_v,
             buf0, buf1, buf2, buf3, acc,
             gsem0, gsem1, gsem2, gsem3, ssem0, ssem1, ssem2, ssem3):
    c = lax.axis_index("c")
    s = lax.axis_index("s")

    bufs = (buf0, buf1, buf2, buf3)
    gsems = (gsem0, gsem1, gsem2, gsem3)
    ssems = (ssem0, ssem1, ssem2, ssem3)

    pltpu.sync_copy(cols3.at[s], cols_v)
    pltpu.sync_copy(rows3.at[s], rows_v)
    pltpu.sync_copy(vals3.at[s], vals_v)

    base_row = s * RPS

    def zero_buf0():
        def zero_body(i, _):
            buf0[pl.ds(i, 1), :] = jnp.zeros((1, H), jnp.float32)
            return 0

        lax.fori_loop(0, B, zero_body, 0)

    def zero_acc():
        for off in range(0, RPS, ZB):
            pltpu.sync_copy(buf0.at[pl.ds(0, ZB)],
                            acc.at[pl.ds(base_row + off, ZB)])

    def gather_start(src, k, j):
        pltpu.async_copy(src.at[cols_v.at[k]], bufs[j], gsems[j])

    def gather_wait(j):
        pltpu.make_async_copy(t1.at[0].at[pl.ds(0, B)], bufs[j], gsems[j]).wait()

    def scatter_start(k, j):
        pltpu.async_copy(bufs[j], acc.at[rows_v.at[k]], ssems[j], add=True)

    def scatter_wait(j):
        pltpu.make_async_copy(bufs[j], acc.at[pl.ds(0, B)], ssems[j]).wait()

    def scale(k, j):
        buf = bufs[j]

        @plsc.parallel_loop(0, B // 16, unroll=2)
        def _group(g):
            vvec = vals_v[k, pl.ds(g * 16, 16)]
            for e in range(16):
                row = g * 16 + e
                v16 = lax.gather(
                    vvec, jnp.full((16, 1), e, jnp.int32),
                    lax.GatherDimensionNumbers(
                        offset_dims=(), collapsed_slice_dims=(0,),
                        start_index_map=(0,)),
                    (1,), mode=lax.GatherScatterMode.PROMISE_IN_BOUNDS)
                for jj in range(H // 16):
                    sl = pl.ds(jj * 16, 16)
                    buf[row, sl] = buf[row, sl] * v16

    def run_round(src):
        gather_start(src, 0, 0)
        gather_start(src, 1, 1)

        def pipe_body(i, _):
            k0 = 4 * i
            for j in range(4):
                k = k0 + j
                nxt = (j + 2) % 4
                if j < 2:
                    pl.when(i > 0)(lambda nxt=nxt: scatter_wait(nxt))
                    gather_start(src, k + 2, nxt)
                else:
                    scatter_wait(nxt)
                    pl.when(i < P - 1)(
                        lambda src=src, k=k, nxt=nxt: gather_start(src, k + 2, nxt))
                gather_wait(j)
                scale(k, j)
                scatter_start(k, j)
            return 0

        lax.fori_loop(0, P, pipe_body, 0)
        scatter_wait(2)
        scatter_wait(3)

    def drain(dst):
        for off in range(0, RPS, ZB):
            pltpu.sync_copy(acc.at[pl.ds(base_row + off, ZB)],
                            dst.at[pl.ds(base_row + off, ZB)])

    zero_buf0()
    zero_acc()
    plsc.subcore_barrier()
    run_round(table.at[c])
    plsc.subcore_barrier()
    drain(t1.at[c])
    zero_buf0()
    zero_acc()
    plsc.subcore_barrier()
    run_round(t1.at[c])
    plsc.subcore_barrier()
    drain(out.at[c])


_BLK = 640


def _mm_body(x_ref, w_ref, o_ref):
    d = jnp.dot(x_ref[...], w_ref[...], preferred_element_type=jnp.float32)
    o_ref[0] = d[:, :H]
    o_ref[1] = d[:, H:]


_mm = pl.pallas_call(
    _mm_body,
    grid=(NP // _BLK,),
    in_specs=[
        pl.BlockSpec((_BLK, C), lambda i: (i, 0)),
        pl.BlockSpec((C, C), lambda i: (0, 0)),
    ],
    out_specs=pl.BlockSpec((NC, _BLK, H), lambda i: (0, i, 0)),
    out_shape=jax.ShapeDtypeStruct((NC, NP, H), jnp.float32),
)


def _fin_body(q_ref, bias_ref, o_ref):
    o_ref[...] = (jnp.concatenate([q_ref[0], q_ref[1]], axis=-1)
                  + bias_ref[...])


_fin = pl.pallas_call(
    _fin_body,
    grid=(NP // _BLK,),
    in_specs=[
        pl.BlockSpec((NC, _BLK, H), lambda i: (0, i, 0)),
        pl.BlockSpec((1, C), lambda i: (0, 0)),
    ],
    out_specs=pl.BlockSpec((_BLK, C), lambda i: (i, 0)),
    out_shape=jax.ShapeDtypeStruct((NP, C), jnp.float32),
)


_PB = 128
_ER = E // _PB
_EPR = EP // _PB


def _prep_body(adj_ref, vals_ref, ro_ref, co_ref, vo_ref):
    zi = jnp.zeros((_EPR - _ER, _PB), jnp.int32)
    zf = jnp.zeros((_EPR - _ER, _PB), jnp.float32)
    ro_ref[...] = jnp.concatenate([adj_ref[0], zi], axis=0)
    co_ref[...] = jnp.concatenate([adj_ref[1], zi], axis=0)
    vo_ref[...] = jnp.concatenate([vals_ref[...], zf], axis=0)


_prep = pl.pallas_call(
    _prep_body,
    grid=(1,),
    in_specs=[
        pl.BlockSpec((2, _ER, _PB), lambda i: (0, 0, 0)),
        pl.BlockSpec((_ER, _PB), lambda i: (0, 0)),
    ],
    out_specs=[
        pl.BlockSpec((_EPR, _PB), lambda i: (0, 0)),
        pl.BlockSpec((_EPR, _PB), lambda i: (0, 0)),
        pl.BlockSpec((_EPR, _PB), lambda i: (0, 0)),
    ],
    out_shape=[
        jax.ShapeDtypeStruct((_EPR, _PB), jnp.int32),
        jax.ShapeDtypeStruct((_EPR, _PB), jnp.int32),
        jax.ShapeDtypeStruct((_EPR, _PB), jnp.float32),
    ],
)


def kernel(adj_index, adj_values, features, weight_matrix, bias):
    adj2 = adj_index.astype(jnp.int32).reshape(2, _ER, _PB)
    vals2 = adj_values.astype(jnp.float32).reshape(_ER, _PB)
    rows2, cols2, vals2p = _prep(adj2, vals2)
    rows3 = rows2.reshape(NS, K, B)
    cols3 = cols2.reshape(NS, K, B)
    vals3 = vals2p.reshape(NS, K, B)

    feats_p = jnp.concatenate(
        [features, jnp.zeros((NP - N, C), jnp.float32)], axis=0)
    t = _mm(feats_p, weight_matrix)
    q, _ = _sc_prop(t, cols3, rows3, vals3)
    return _fin(q, bias)[:N]
